# Initial kernel scaffold; baseline (speedup 1.0000x reference)
import jax
import jax.numpy as jnp
from jax.experimental import pallas as pl


def kernel(h_video, h_tag, h0_video, h0_tag, fb_src, fb_dst, st_src, st_dst, ht_src, ht_dst, W_k_video, b_k_video, W_k_tag, b_k_tag, W_q_video, b_q_video, W_q_tag, b_q_tag, relation_pri, relation_att):
    raise NotImplementedError("write your pallas kernel here")


# speedup vs baseline: 6.0134x; 6.0134x over previous
import functools
import math

import jax
import jax.numpy as jnp
from jax import lax
from jax.experimental import pallas as pl
from jax.experimental.pallas import tpu as pltpu
from jax.experimental.pallas import tpu_sc as plsc

H = 8
DK = 16
D = 128
NV = 50000
NT = 10000
EFB = 250000
EST = 50000
EHT = 250000
SQRT_DK = math.sqrt(DK)

C = 128
NWS = 16
L = 16

NVP = 50176
NTP = 10240
RV = NVP // NWS
RT = NTP // NWS

_GRAN = NWS * C
EFBP = ((EFB + _GRAN - 1) // _GRAN) * _GRAN
ESTP = ((EST + _GRAN - 1) // _GRAN) * _GRAN
EHTP = EFBP

_MESH = plsc.VectorSubcoreMesh(core_axis_name="c", subcore_axis_name="s")


def _mm2_body(x_ref, m0_ref, c0_ref, m1_ref, c1_ref, o0_ref, o1_ref):
    x = x_ref[...]
    o0_ref[...] = jnp.dot(x, m0_ref[...], preferred_element_type=jnp.float32) + c0_ref[...]
    o1_ref[...] = jnp.dot(x, m1_ref[...], preferred_element_type=jnp.float32) + c1_ref[...]


def _mm1_body(x_ref, m_ref, c_ref, o_ref):
    o_ref[...] = jnp.dot(x_ref[...], m_ref[...], preferred_element_type=jnp.float32) + c_ref[...]


def _proj2(x, m0, c0, m1, c1, blk):
    n = x.shape[0]
    out = jax.ShapeDtypeStruct((n, D), jnp.float32)
    return pl.pallas_call(
        _mm2_body,
        grid=(n // blk,),
        in_specs=[
            pl.BlockSpec((blk, D), lambda i: (i, 0)),
            pl.BlockSpec((D, D), lambda i: (0, 0)),
            pl.BlockSpec((1, D), lambda i: (0, 0)),
            pl.BlockSpec((D, D), lambda i: (0, 0)),
            pl.BlockSpec((1, D), lambda i: (0, 0)),
        ],
        out_specs=[pl.BlockSpec((blk, D), lambda i: (i, 0)),
                   pl.BlockSpec((blk, D), lambda i: (i, 0))],
        out_shape=[out, out],
    )(x, m0, c0.reshape(1, D), m1, c1.reshape(1, D))


def _proj1(x, m, c, blk):
    n = x.shape[0]
    return pl.pallas_call(
        _mm1_body,
        grid=(n // blk,),
        in_specs=[
            pl.BlockSpec((blk, D), lambda i: (i, 0)),
            pl.BlockSpec((D, D), lambda i: (0, 0)),
            pl.BlockSpec((1, D), lambda i: (0, 0)),
        ],
        out_specs=pl.BlockSpec((blk, D), lambda i: (i, 0)),
        out_shape=jax.ShapeDtypeStruct((n, D), jnp.float32),
    )(x, m, c.reshape(1, D))


def _take16(v, idx):
    return lax.gather(
        v, idx[:, None],
        lax.GatherDimensionNumbers(offset_dims=(), collapsed_slice_dims=(0,),
                                   start_index_map=(0,)),
        (1,), mode=lax.GatherScatterMode.PROMISE_IN_BOUNDS)


@functools.partial(
    pl.kernel,
    mesh=_MESH,
    out_type=[
        jax.ShapeDtypeStruct((EFBP * H,), jnp.float32),
        jax.ShapeDtypeStruct((ESTP * H,), jnp.float32),
        jax.ShapeDtypeStruct((EHTP * H,), jnp.float32),
        jax.ShapeDtypeStruct((EFBP * H,), jnp.float32),
        jax.ShapeDtypeStruct((ESTP * H,), jnp.float32),
        jax.ShapeDtypeStruct((EHTP * H,), jnp.float32),
        jax.ShapeDtypeStruct((NVP // 8, 128), jnp.float32),
        jax.ShapeDtypeStruct((NTP // 8, 128), jnp.float32),
    ],
    scratch_types=[
        pltpu.VMEM((C,), jnp.int32),
        pltpu.VMEM((C,), jnp.int32),
        pltpu.VMEM((C, D), jnp.float32),
        pltpu.VMEM((C, D), jnp.float32),
        pltpu.VMEM((C * H,), jnp.float32),
        pltpu.VMEM((RV // 8 + 1, 128), jnp.float32),
        pltpu.VMEM((C, 128), jnp.float32),
        pltpu.SemaphoreType.DMA,
        pltpu.SemaphoreType.DMA,
    ],
)
def _edge_kernel(kfb, qv, kst, kht, qt, fb_s, fb_d, st_s, st_d, ht_s, ht_d,
                 o_fb, o_st, o_ht, e_fb, e_st, e_ht, s_v, s_t,
                 sidx, didx, krows, qrows, tbuf, acc, grows, sem_k, sem_q):
    cid = lax.axis_index("c")
    sid = lax.axis_index("s")

    lane = lax.iota(jnp.int32, L)
    lo_m = lane < 8
    x8 = lane ^ 8
    x4 = lane ^ 4
    x2 = lane ^ 2
    x1 = lane ^ 1
    hmasks = [(lane & 7) == h for h in range(H)]
    hi_idx = 8 + (lane & 7)
    lo_idx = lane & 7

    def score_chunk(ci, src_hbm, dst_hbm, ktab, qtab, eout, e_true):
        base = ci * C
        pltpu.sync_copy(src_hbm.at[pl.ds(base, C)], sidx)
        pltpu.sync_copy(dst_hbm.at[pl.ds(base, C)], didx)
        cp_k = pltpu.async_copy(ktab.at[sidx], krows, sem_k)
        cp_q = pltpu.async_copy(qtab.at[didx], qrows, sem_q)
        cp_k.wait()
        cp_q.wait()

        def pair(i, _):
            ea = 2 * i
            eb = 2 * i + 1
            w = jnp.zeros((L,), jnp.float32)
            for h in range(H):
                pa = krows[ea, pl.ds(h * DK, DK)] * qrows[ea, pl.ds(h * DK, DK)]
                pb = krows[eb, pl.ds(h * DK, DK)] * qrows[eb, pl.ds(h * DK, DK)]
                fa = pa + _take16(pa, x8)
                fb = pb + _take16(pb, x8)
                m = jnp.where(lo_m, fa, fb)
                m = m + _take16(m, x4)
                m = m + _take16(m, x2)
                m = m + _take16(m, x1)
                w = jnp.where(hmasks[h], m, w)
            ex = jnp.exp(w)
            gmask = (base * H + i * L + lane) < (e_true * H)
            ex = jnp.where(gmask, ex, 0.0)
            tbuf[pl.ds(i * L, L)] = ex
            return 0

        lax.fori_loop(0, C // 2, pair, 0)
        pltpu.sync_copy(tbuf, eout.at[pl.ds(base * H, C * H)])

    def sum_chunk(ci, dst_hbm, ein, lo, nrange):
        base = ci * C
        pltpu.sync_copy(dst_hbm.at[pl.ds(base, C)], didx)
        pltpu.sync_copy(ein.at[pl.ds(base * H, C * H)], tbuf)

        def group(g, _):
            dv = didx[pl.ds(g * L, L)] - lo
            for p in range(L // 2):
                ex = tbuf[pl.ds((g * (L // 2) + p) * L, L)]
                row_a = jnp.where(lo_m, ex, 0.0)
                row_b = jnp.where(lo_m, _take16(ex, hi_idx), 0.0)
                ra = dv[2 * p]
                rb = dv[2 * p + 1]
                ra = jnp.where((ra >= 0) & (ra < nrange), ra, nrange)
                rb = jnp.where((rb >= 0) & (rb < nrange), rb, nrange)
                va = acc[ra >> 3, pl.ds((ra & 7) * L, L)]
                acc[ra >> 3, pl.ds((ra & 7) * L, L)] = va + row_a
                vb = acc[rb >> 3, pl.ds((rb & 7) * L, L)]
                acc[rb >> 3, pl.ds((rb & 7) * L, L)] = vb + row_b
            return 0

        lax.fori_loop(0, C // L, group, 0)

    def scan_all(dst_hbm, ein, e_pad, lo, nrange):
        def body(ci, _):
            sum_chunk(ci, dst_hbm, ein, lo, nrange)
            return 0
        lax.fori_loop(0, e_pad // C, body, 0)

    def norm_chunk(ci, dst_hbm, ein, s_hbm, oout):
        base = ci * C
        pltpu.sync_copy(dst_hbm.at[pl.ds(base, C)], didx)
        pltpu.sync_copy(ein.at[pl.ds(base * H, C * H)], tbuf)
        for j in range(C // L):
            sidx[pl.ds(j * L, L)] = didx[pl.ds(j * L, L)] >> 3
        pltpu.async_copy(s_hbm.at[sidx], grows, sem_k).wait()

        def group(g, _):
            dv = didx[pl.ds(g * L, L)] & 7
            for p in range(L // 2):
                ea = g * L + 2 * p
                eb = ea + 1
                ev = tbuf[pl.ds((g * (L // 2) + p) * L, L)]
                sa = grows[ea, pl.ds(dv[2 * p] * L, L)]
                sb = grows[eb, pl.ds(dv[2 * p + 1] * L, L)]
                s = jnp.where(lo_m, sa, _take16(sb, lo_idx))
                tbuf[pl.ds((g * (L // 2) + p) * L, L)] = ev / s
            return 0

        lax.fori_loop(0, C // L, group, 0)
        pltpu.sync_copy(tbuf, oout.at[pl.ds(base * H, C * H)])

    def loop_chunks(fn, e_pad, *args):
        def body(t, _):
            fn(sid + t * NWS, *args)
            return 0

        lax.fori_loop(0, e_pad // C // NWS, body, 0)

    @pl.when(cid == 0)
    def _():
        loop_chunks(score_chunk, EFBP, fb_s, fb_d, kfb, qv, e_fb, EFB)

    @pl.when(cid == 1)
    def _():
        loop_chunks(score_chunk, ESTP, st_s, st_d, kst, qt, e_st, EST)
        loop_chunks(score_chunk, EHTP, ht_s, ht_d, kht, qt, e_ht, EHT)

    plsc.subcore_barrier()

    def _za(i, _):
        for j in range(8):
            acc[i, pl.ds(j * L, L)] = jnp.zeros((L,), jnp.float32)
        return 0
    lax.fori_loop(0, RV // 8 + 1, _za, 0)

    @pl.when(cid == 0)
    def _():
        lo = sid * RV
        scan_all(fb_d, e_fb, EFBP, lo, RV)
        pltpu.sync_copy(acc.at[pl.ds(0, RV // 8)], s_v.at[pl.ds(sid * (RV // 8), RV // 8)])

    @pl.when(cid == 1)
    def _():
        lo = sid * RT
        scan_all(st_d, e_st, ESTP, lo, RT)
        scan_all(ht_d, e_ht, EHTP, lo, RT)
        pltpu.sync_copy(acc.at[pl.ds(0, RT // 8)], s_t.at[pl.ds(sid * (RT // 8), RT // 8)])

    plsc.subcore_barrier()

    @pl.when(cid == 0)
    def _():
        loop_chunks(norm_chunk, EFBP, fb_d, e_fb, s_v, o_fb)

    @pl.when(cid == 1)
    def _():
        loop_chunks(norm_chunk, ESTP, st_d, e_st, s_t, o_st)
        loop_chunks(norm_chunk, EHTP, ht_d, e_ht, s_t, o_ht)


def _pad_edges(x, n):
    return jnp.pad(x, (0, n - x.shape[0]))


def kernel(h_video, h_tag, h0_video, h0_tag, fb_src, fb_dst, st_src, st_dst,
           ht_src, ht_dst, W_k_video, b_k_video, W_k_tag, b_k_tag,
           W_q_video, b_q_video, W_q_tag, b_q_tag, relation_pri, relation_att):
    scale = relation_pri / SQRT_DK
    att_s = relation_att * scale[:, :, None, None]
    B = jnp.zeros((3, D, D), jnp.float32)
    for h in range(H):
        B = B.at[:, DK * h:DK * (h + 1), DK * h:DK * (h + 1)].set(att_s[:, h])
    M_fb = W_k_video.T @ B[0]
    c_fb = b_k_video @ B[0]
    M_st = W_k_tag.T @ B[1]
    c_st = b_k_tag @ B[1]
    M_ht = W_k_video.T @ B[2]
    c_ht = b_k_video @ B[2]

    K_fb, K_ht = _proj2(h_video, M_fb, c_fb, M_ht, c_ht, blk=1000)
    Q_v = _proj1(h0_video, W_q_video.T, b_q_video, blk=1000)
    K_st = _proj1(h_tag, M_st, c_st, blk=1000)
    Q_t = _proj1(h0_tag, W_q_tag.T, b_q_tag, blk=1000)

    fb_s = _pad_edges(fb_src, EFBP)
    fb_d = _pad_edges(fb_dst, EFBP)
    st_s = _pad_edges(st_src, ESTP)
    st_d = _pad_edges(st_dst, ESTP)
    ht_s = _pad_edges(ht_src, EHTP)
    ht_d = _pad_edges(ht_dst, EHTP)

    o_fb, o_st, o_ht, _, _, _, _, _ = _edge_kernel(
        K_fb, Q_v, K_st, K_ht, Q_t, fb_s, fb_d, st_s, st_d, ht_s, ht_d)

    return jnp.concatenate([
        o_fb.reshape(EFBP, H)[:EFB],
        o_st.reshape(ESTP, H)[:EST],
        o_ht.reshape(EHTP, H)[:EHT],
    ], axis=0)
